# Initial kernel scaffold; baseline (speedup 1.0000x reference)
import jax
import jax.numpy as jnp
from jax.experimental import pallas as pl


def kernel(src, index, out):
    raise NotImplementedError("write your pallas kernel here")


# speedup vs baseline: 3.7163x; 3.7163x over previous
import functools

import jax
import jax.numpy as jnp
from jax import lax
from jax.experimental import pallas as pl
from jax.experimental.pallas import tpu as pltpu
from jax.experimental.pallas import tpu_sc as plsc

N_EDGES = 320000
N_NODES = 10000
D_FEAT = 128

NC = 2
NS = 16
NW = NC * NS

EPT = N_EDGES // NW
EBLK = 80
NBLK = EPT // EBLK

N_PAD = 10240
RPT = N_PAD // NS
ZCH = 128

_mesh = plsc.VectorSubcoreMesh(core_axis_name="c", subcore_axis_name="s")


@functools.partial(
    pl.kernel,
    mesh=_mesh,
    out_type=jax.ShapeDtypeStruct((NC, N_PAD, D_FEAT), jnp.float32),
    scratch_types=[
        pltpu.VMEM((EBLK,), jnp.int32),
        pltpu.VMEM((EBLK, D_FEAT), jnp.float32),
        pltpu.VMEM((ZCH, D_FEAT), jnp.float32),
        pltpu.VMEM_SHARED((N_PAD, D_FEAT), jnp.float32),
    ],
)
def _sc_scatter_add(src, index, part, idx_v, blk_v, zbuf, acc):
    cid = lax.axis_index("c")
    sid = lax.axis_index("s")
    tid = cid * NS + sid

    zero = jnp.zeros((16,), jnp.float32)

    def zero_body(i, carry):
        r = i // (D_FEAT // 16)
        c = i % (D_FEAT // 16)
        zbuf[r, pl.ds(c * 16, 16)] = zero
        return carry

    lax.fori_loop(0, ZCH * (D_FEAT // 16), zero_body, 0)

    def zcopy_body(j, carry):
        pltpu.sync_copy(zbuf, acc.at[pl.ds(sid * RPT + j * ZCH, ZCH)])
        return carry

    lax.fori_loop(0, RPT // ZCH, zcopy_body, 0)
    plsc.subcore_barrier()

    base = tid * EPT

    def body(b, carry):
        off = base + b * EBLK
        pltpu.sync_copy(index.at[pl.ds(off, EBLK)], idx_v)
        pltpu.sync_copy(src.at[pl.ds(off, EBLK)], blk_v)
        pltpu.sync_copy(blk_v, acc.at[idx_v], add=True)
        return carry

    lax.fori_loop(0, NBLK, body, 0)
    plsc.subcore_barrier()

    pltpu.sync_copy(acc.at[pl.ds(sid * RPT, RPT)],
                    part.at[cid, pl.ds(sid * RPT, RPT)])


def _combine_body(p_ref, o_ref, r_ref):
    r_ref[...] = p_ref[0] + p_ref[1] + o_ref[...]


def _combine(part, out):
    rows = 1000
    return pl.pallas_call(
        _combine_body,
        grid=(N_NODES // rows,),
        in_specs=[
            pl.BlockSpec((NC, rows, D_FEAT), lambda i: (0, i, 0)),
            pl.BlockSpec((rows, D_FEAT), lambda i: (i, 0)),
        ],
        out_specs=pl.BlockSpec((rows, D_FEAT), lambda i: (i, 0)),
        out_shape=jax.ShapeDtypeStruct((N_NODES, D_FEAT), jnp.float32),
    )(part, out)


@jax.jit
def kernel(src, index, out):
    part = _sc_scatter_add(src, index.astype(jnp.int32))
    return _combine(part, out)


# speedup vs baseline: 5.8575x; 1.5762x over previous
import functools

import jax
import jax.numpy as jnp
from jax import lax
from jax.experimental import pallas as pl
from jax.experimental.pallas import tpu as pltpu
from jax.experimental.pallas import tpu_sc as plsc

N_EDGES = 320000
N_NODES = 10000
D_FEAT = 128

NC = 2
NS = 16
NW = NC * NS

EPT = N_EDGES // NW
EBLK = 80
NBLK = EPT // EBLK

N_PAD = 10240
RPT = N_PAD // NS
IRT = 624

_mesh = plsc.VectorSubcoreMesh(core_axis_name="c", subcore_axis_name="s")


@functools.partial(
    pl.kernel,
    mesh=_mesh,
    out_type=jax.ShapeDtypeStruct((NC, N_PAD, D_FEAT), jnp.float32),
    scratch_types=[
        pltpu.VMEM((EBLK,), jnp.int32),
        pltpu.VMEM((EBLK,), jnp.int32),
        pltpu.VMEM((EBLK, D_FEAT), jnp.float32),
        pltpu.VMEM((EBLK, D_FEAT), jnp.float32),
        pltpu.VMEM_SHARED((N_PAD, D_FEAT), jnp.float32),
        pltpu.SemaphoreType.DMA,
        pltpu.SemaphoreType.DMA,
    ],
)
def _sc_scatter_add(src, index, out, part, idx0, idx1, blk0, blk1, acc,
                    sem0, sem1):
    cid = lax.axis_index("c")
    sid = lax.axis_index("s")
    tid = cid * NS + sid

    @pl.when(sid < NS - 1)
    def _():
        r0 = pl.multiple_of(sid * IRT, 8)
        pltpu.sync_copy(out.at[pl.ds(r0, IRT)], acc.at[pl.ds(r0, IRT)])

    @pl.when(sid == NS - 1)
    def _():
        pltpu.sync_copy(out.at[pl.ds((NS - 1) * IRT, 640)],
                        acc.at[pl.ds((NS - 1) * IRT, 640)])

    plsc.subcore_barrier()

    base = tid * EPT

    def start(b, idxb, blkb, sem):
        off = pl.multiple_of(base + b * EBLK, 8)
        pltpu.make_async_copy(index.at[pl.ds(off, EBLK)], idxb, sem).start()
        pltpu.make_async_copy(src.at[pl.ds(off, EBLK)], blkb, sem).start()

    def wait(idxb, blkb, sem):
        pltpu.make_async_copy(index.at[pl.ds(base, EBLK)], idxb, sem).wait()
        pltpu.make_async_copy(src.at[pl.ds(base, EBLK)], blkb, sem).wait()

    def scat(idxb, blkb):
        pltpu.sync_copy(blkb, acc.at[idxb], add=True)

    start(0, idx0, blk0, sem0)

    def body(i, carry):
        b = 2 * i
        wait(idx0, blk0, sem0)
        start(b + 1, idx1, blk1, sem1)
        scat(idx0, blk0)
        wait(idx1, blk1, sem1)
        start(b + 2, idx0, blk0, sem0)
        scat(idx1, blk1)
        return carry

    lax.fori_loop(0, (NBLK - 1) // 2, body, 0)
    wait(idx0, blk0, sem0)
    scat(idx0, blk0)
    plsc.subcore_barrier()

    r0 = pl.multiple_of(sid * RPT, 8)
    pltpu.sync_copy(acc.at[pl.ds(r0, RPT)], part.at[cid, pl.ds(r0, RPT)])


def _combine_body(p_ref, o_ref, r_ref):
    r_ref[...] = p_ref[0] + p_ref[1] - o_ref[...]


def _combine(part, out):
    rows = 1000
    return pl.pallas_call(
        _combine_body,
        grid=(N_NODES // rows,),
        in_specs=[
            pl.BlockSpec((NC, rows, D_FEAT), lambda i: (0, i, 0)),
            pl.BlockSpec((rows, D_FEAT), lambda i: (i, 0)),
        ],
        out_specs=pl.BlockSpec((rows, D_FEAT), lambda i: (i, 0)),
        out_shape=jax.ShapeDtypeStruct((N_NODES, D_FEAT), jnp.float32),
    )(part, out)


@jax.jit
def kernel(src, index, out):
    part = _sc_scatter_add(src, index.astype(jnp.int32), out)
    return _combine(part, out)


# speedup vs baseline: 8.0665x; 1.3771x over previous
import functools

import jax
import jax.numpy as jnp
from jax import lax
from jax.experimental import pallas as pl
from jax.experimental.pallas import tpu as pltpu
from jax.experimental.pallas import tpu_sc as plsc

N_EDGES = 320000
N_NODES = 10000
D_FEAT = 128

NC = 2
NS = 16
NW = NC * NS

EPT = N_EDGES // NW
EBLK = 80
NBLK = EPT // EBLK
DEPTH = 4

N_PAD = 10240
RPT = N_PAD // NS
IRT = 624

_mesh = plsc.VectorSubcoreMesh(core_axis_name="c", subcore_axis_name="s")


@functools.partial(
    pl.kernel,
    mesh=_mesh,
    out_type=jax.ShapeDtypeStruct((NC, N_PAD, D_FEAT), jnp.float32),
    scratch_types=(
        [pltpu.VMEM((EBLK,), jnp.int32) for _ in range(DEPTH)]
        + [pltpu.VMEM((EBLK, D_FEAT), jnp.float32) for _ in range(DEPTH)]
        + [pltpu.VMEM_SHARED((N_PAD, D_FEAT), jnp.float32)]
        + [pltpu.SemaphoreType.DMA for _ in range(2 * DEPTH)]
    ),
)
def _sc_scatter_add(src, index, out, part, *refs):
    idxs = refs[0:DEPTH]
    blks = refs[DEPTH:2 * DEPTH]
    acc = refs[2 * DEPTH]
    lsems = refs[2 * DEPTH + 1:3 * DEPTH + 1]
    ssems = refs[3 * DEPTH + 1:4 * DEPTH + 1]
    cid = lax.axis_index("c")
    sid = lax.axis_index("s")
    tid = cid * NS + sid

    @pl.when(sid < NS - 1)
    def _():
        r0 = pl.multiple_of(sid * IRT, 8)
        pltpu.sync_copy(out.at[pl.ds(r0, IRT)], acc.at[pl.ds(r0, IRT)])

    @pl.when(sid == NS - 1)
    def _():
        pltpu.sync_copy(out.at[pl.ds((NS - 1) * IRT, 640)],
                        acc.at[pl.ds((NS - 1) * IRT, 640)])

    plsc.subcore_barrier()

    base = tid * EPT

    def start_load(b, p):
        off = pl.multiple_of(base + b * EBLK, 8)
        pltpu.make_async_copy(index.at[pl.ds(off, EBLK)], idxs[p],
                              lsems[p]).start()
        pltpu.make_async_copy(src.at[pl.ds(off, EBLK)], blks[p],
                              lsems[p]).start()

    def wait_load(p):
        pltpu.make_async_copy(index.at[pl.ds(base, EBLK)], idxs[p],
                              lsems[p]).wait()
        pltpu.make_async_copy(src.at[pl.ds(base, EBLK)], blks[p],
                              lsems[p]).wait()

    def start_scat(p):
        pltpu.make_async_copy(blks[p], acc.at[idxs[p]],
                              ssems[p]).start(add=True)

    def wait_scat(p):
        pltpu.make_async_copy(blks[p], acc.at[idxs[p]], ssems[p]).wait()

    for p in range(DEPTH - 1):
        start_load(p, p)

    def body(i, carry):
        for p in range(DEPTH):
            b = DEPTH * i + p
            wait_load(p)
            start_scat(p)
            q = (p + DEPTH - 1) % DEPTH
            if p == 0:
                @pl.when(i > 0)
                def _():
                    wait_scat(q)
            else:
                wait_scat(q)

            @pl.when(b + DEPTH - 1 < NBLK)
            def _():
                start_load(b + DEPTH - 1, q)
        return carry

    lax.fori_loop(0, NBLK // DEPTH, body, 0)
    p_last = NBLK - 1 - DEPTH * (NBLK // DEPTH)
    wait_load(p_last)
    start_scat(p_last)
    wait_scat((p_last + DEPTH - 1) % DEPTH)
    wait_scat(p_last)
    plsc.subcore_barrier()

    r0 = pl.multiple_of(sid * RPT, 8)
    pltpu.sync_copy(acc.at[pl.ds(r0, RPT)], part.at[cid, pl.ds(r0, RPT)])


def _combine_body(p_ref, o_ref, r_ref):
    r_ref[...] = p_ref[0] + p_ref[1] - o_ref[...]


def _combine(part, out):
    rows = 1000
    return pl.pallas_call(
        _combine_body,
        grid=(N_NODES // rows,),
        in_specs=[
            pl.BlockSpec((NC, rows, D_FEAT), lambda i: (0, i, 0)),
            pl.BlockSpec((rows, D_FEAT), lambda i: (i, 0)),
        ],
        out_specs=pl.BlockSpec((rows, D_FEAT), lambda i: (i, 0)),
        out_shape=jax.ShapeDtypeStruct((N_NODES, D_FEAT), jnp.float32),
    )(part, out)


@jax.jit
def kernel(src, index, out):
    part = _sc_scatter_add(src, index.astype(jnp.int32), out)
    return _combine(part, out)
